# Initial kernel scaffold; baseline (speedup 1.0000x reference)
import jax
import jax.numpy as jnp
from jax.experimental import pallas as pl


def kernel(hidden, W_pool, b_pool, W_a1, b_a1, W_a2, b_a2, W_s1, b_s1, W_s2, b_s2):
    raise NotImplementedError("write your pallas kernel here")


# speedup vs baseline: 7.1779x; 7.1779x over previous
import functools

import jax
import jax.numpy as jnp
from jax import lax
from jax.experimental import pallas as pl
from jax.experimental.pallas import tpu as pltpu
from jax.experimental.pallas import tpu_sc as plsc

NB = 4
BS = 8
SEQ = 32
H = 64
MS = 6
TILE = 256
GCHUNK = 512


def _select_body(hid_ref, Wp, bp, Wa1, ba1, Wa2, ba2, Ws1, bs1, Ws2, bs2,
                 sel_ref, alloc_ref):
    T = TILE
    TB = T * NB
    f32, i32 = jnp.float32, jnp.int32

    def expand(x):
        return jnp.broadcast_to(x[:, None, :], (T, NB, 1)).reshape(TB, 1)

    def gsum(x):
        return jnp.sum(x.reshape(T, NB, 1), axis=1)

    h = hid_ref[...]

    s1 = jnp.maximum(
        jnp.dot(h, Ws1[...], preferred_element_type=f32) + bs1[...], 0.0)
    s13 = s1.reshape(TB, BS, H // 2)
    cols = [jnp.dot(s13[:, j, :], Ws2[...], preferred_element_type=f32)
            for j in range(BS)]
    scores = jnp.concatenate(cols, axis=1) + bs2[...]

    pm = jnp.mean(h.reshape(TB, BS, H), axis=1)
    pooled = jnp.dot(pm, Wp[...], preferred_element_type=f32) + bp[...]
    p3 = pooled.reshape(T, NB, H)
    acc = jnp.zeros((T, H), f32) + ba1[...]
    for b in range(NB):
        acc = acc + jnp.dot(p3[:, b, :], Wa1[b * H:(b + 1) * H, :],
                            preferred_element_type=f32)
    h1 = jnp.maximum(acc, 0.0)
    h1e = jnp.broadcast_to(h1[:, None, :], (T, NB, H)).reshape(TB, H)
    lmat = jnp.dot(h1e, Wa2[...], preferred_element_type=f32) + ba2[...]
    rowb = lax.broadcasted_iota(i32, (TB, NB), 0) % NB
    col4 = lax.broadcasted_iota(i32, (TB, NB), 1)
    lcol = jnp.sum(jnp.where(rowb == col4, lmat, 0.0), axis=1,
                   keepdims=True)

    m = expand(jnp.max(lcol.reshape(T, NB, 1), axis=1))
    e = jnp.exp(lcol - m)
    soft = e / expand(gsum(e)) * MS
    fl = jnp.floor(soft)
    fr = soft - fl
    fli = fl.astype(i32)
    rem = expand(MS - gsum(fli))

    bidx = rowb[:, 0:1]
    frg = fr.reshape(T, NB, 1)
    rank4 = jnp.zeros((TB, 1), i32)
    for j in range(NB):
        fj = expand(frg[:, j, :])
        beat = (fj > fr) | ((fj == fr) & (bidx > j))
        rank4 = rank4 + beat.astype(i32)
    ia = fli + (rank4 < rem).astype(i32)
    alloc_ref[...] = ia.astype(f32)

    iag = ia.reshape(T, NB, 1)
    off = jnp.zeros((TB, 1), i32)
    for j in range(NB - 1):
        off = off + jnp.where(bidx > j, expand(iag[:, j, :]), 0)

    lane8 = lax.broadcasted_iota(i32, (1, BS), 1)
    rankt = jnp.zeros((TB, BS), i32)
    for j in range(BS):
        sj = scores[:, j:j + 1]
        beat = (sj > scores) | ((sj == scores) & (lane8 > j))
        rankt = rankt + beat.astype(i32)

    valid = rankt < ia
    slot = off + rankt
    tloc = lane8 + BS * bidx

    sel_cols = []
    for s in range(MS):
        msk = valid & (slot == s)
        c = jnp.sum(jnp.where(msk, tloc, 0), axis=1, keepdims=True)
        sel_cols.append(gsum(c))
    sel_local = jnp.concatenate(sel_cols, axis=1)
    rows = (pl.program_id(0) * T
            + lax.broadcasted_iota(i32, (T, 1), 0))
    sel_ref[...] = rows * SEQ + sel_local


def _run_select(hid2d, Wp, bp, Wa1, ba1, Wa2, ba2, Ws1, bs1, Ws2, bs2, Bsz):
    grid = Bsz // TILE
    full = lambda shape: pl.BlockSpec(shape, lambda i: (0, 0))
    return pl.pallas_call(
        _select_body,
        grid=(grid,),
        in_specs=[
            pl.BlockSpec((TILE * SEQ, H), lambda i: (i, 0)),
            full((H, H)), full((1, H)),
            full((NB * H, H)), full((1, H)),
            full((H, NB)), full((1, NB)),
            full((H, H // 2)), full((1, H // 2)),
            full((H // 2, 1)), full((1, 1)),
        ],
        out_specs=[
            pl.BlockSpec((TILE, MS), lambda i: (i, 0)),
            pl.BlockSpec((TILE * NB, 1), lambda i: (i, 0)),
        ],
        out_shape=[
            jax.ShapeDtypeStruct((Bsz, MS), jnp.int32),
            jax.ShapeDtypeStruct((Bsz * NB, 1), jnp.float32),
        ],
    )(hid2d, Wp, bp, Wa1, ba1, Wa2, ba2, Ws1, bs1, Ws2, bs2)


def _run_gather(hid2d, sel_flat):
    n_rows = sel_flat.shape[0]
    info = plsc.get_sparse_core_info()
    nw = info.num_cores * info.num_subcores
    per_w = n_rows // nw
    chunks = per_w // GCHUNK
    mesh = plsc.VectorSubcoreMesh(core_axis_name="c", subcore_axis_name="s")

    @functools.partial(
        pl.kernel,
        out_type=jax.ShapeDtypeStruct((n_rows, H), jnp.float32),
        mesh=mesh,
        scratch_types=[
            pltpu.VMEM((GCHUNK,), jnp.int32),
            pltpu.VMEM((GCHUNK, H), jnp.float32),
            pltpu.SemaphoreType.DMA,
        ],
        compiler_params=pltpu.CompilerParams(use_tc_tiling_on_sc=False),
    )
    def gk(hid_hbm, sel_hbm, out_hbm, idx_v, rows_v, sem):
        wid = lax.axis_index("s") * info.num_cores + lax.axis_index("c")
        base0 = wid * per_w
        for c in range(chunks):
            base = base0 + c * GCHUNK
            pltpu.sync_copy(sel_hbm.at[pl.ds(base, GCHUNK)], idx_v)
            pltpu.async_copy(hid_hbm.at[idx_v], rows_v, sem).wait()
            pltpu.sync_copy(rows_v, out_hbm.at[pl.ds(base, GCHUNK)])

    return gk(hid2d, sel_flat)


def kernel(hidden, W_pool, b_pool, W_a1, b_a1, W_a2, b_a2, W_s1, b_s1, W_s2, b_s2):
    Bsz = hidden.shape[0]
    hid2d = hidden.reshape(Bsz * SEQ, H)
    sel, ia_f = _run_select(
        hid2d, W_pool, b_pool.reshape(1, H), W_a1, b_a1.reshape(1, H),
        W_a2, b_a2.reshape(1, NB), W_s1, b_s1.reshape(1, H // 2),
        W_s2, b_s2.reshape(1, 1), Bsz)
    mem_flat = _run_gather(hid2d, sel.reshape(Bsz * MS))
    memory = mem_flat.reshape(Bsz, MS, H)
    mask = jnp.ones((Bsz, MS), hidden.dtype)
    return memory, mask, ia_f.reshape(Bsz, NB)


# speedup vs baseline: 8.7608x; 1.2205x over previous
import functools

import jax
import jax.numpy as jnp
from jax import lax
from jax.experimental import pallas as pl
from jax.experimental.pallas import tpu as pltpu
from jax.experimental.pallas import tpu_sc as plsc

NB = 4
BS = 8
SEQ = 32
H = 64
MS = 6
TILE = 256
GCHUNK = 512


def _select_body(hid_ref, Wp, bp, Wa1, ba1, Wa2, ba2, Ws1, bs1, Ws2, bs2,
                 R4, R4T, sel_ref, alloc_ref):
    T = TILE
    TB = T * NB
    f32, i32 = jnp.float32, jnp.int32
    h = hid_ref[...]

    s1 = jnp.maximum(
        jnp.dot(h, Ws1[...], preferred_element_type=f32) + bs1[...], 0.0)
    s13 = s1.reshape(TB, BS, H // 2)
    cols = [jnp.dot(s13[:, j, :], Ws2[...], preferred_element_type=f32)
            for j in range(BS)]
    scores = jnp.concatenate(cols, axis=1) + bs2[...]

    pm = jnp.mean(h.reshape(TB, BS, H), axis=1)
    pooled = jnp.dot(pm, Wp[...], preferred_element_type=f32) + bp[...]
    p3 = pooled.reshape(T, NB, H)
    acc = jnp.zeros((T, H), f32) + ba1[...]
    for b in range(NB):
        acc = acc + jnp.dot(p3[:, b, :], Wa1[b * H:(b + 1) * H, :],
                            preferred_element_type=f32)
    h1 = jnp.maximum(acc, 0.0)
    logits = jnp.dot(h1, Wa2[...], preferred_element_type=f32) + ba2[...]

    m = jnp.max(logits, axis=1, keepdims=True)
    e = jnp.exp(logits - m)
    soft = e / jnp.sum(e, axis=1, keepdims=True) * MS
    fl = jnp.floor(soft)
    fr = soft - fl
    fli = fl.astype(i32)
    rem = MS - jnp.sum(fli, axis=1, keepdims=True)

    lane4 = lax.broadcasted_iota(i32, (1, NB), 1)
    rank4 = jnp.zeros((T, NB), i32)
    for j in range(NB):
        fj = fr[:, j:j + 1]
        beat = (fj > fr) | ((fj == fr) & (lane4 > j))
        rank4 = rank4 + beat.astype(i32)
    ia = fli + (rank4 < rem).astype(i32)
    alloc_ref[...] = ia.astype(f32)

    off = jnp.zeros((T, NB), i32)
    for j in range(NB - 1):
        off = off + jnp.where(lane4 > j, ia[:, j:j + 1], 0)

    ko = jnp.concatenate([ia.astype(f32), off.astype(f32)], axis=1)
    koexp = jnp.dot(R4[...], ko, preferred_element_type=f32)
    bidx = lax.broadcasted_iota(i32, (TB, 1), 0) % NB
    col8 = lax.broadcasted_iota(i32, (TB, 2 * NB), 1)
    kcol = jnp.sum(jnp.where(col8 == bidx, koexp, 0.0), axis=1,
                   keepdims=True).astype(i32)
    offcol = jnp.sum(jnp.where(col8 == bidx + NB, koexp, 0.0), axis=1,
                     keepdims=True).astype(i32)

    lane8 = lax.broadcasted_iota(i32, (1, BS), 1)
    rankt = jnp.zeros((TB, BS), i32)
    for j in range(BS):
        sj = scores[:, j:j + 1]
        beat = (sj > scores) | ((sj == scores) & (lane8 > j))
        rankt = rankt + beat.astype(i32)

    valid = rankt < kcol
    slot = offcol + rankt
    tloc = (lane8 + BS * bidx).astype(f32)

    cols = [jnp.where(valid & (slot == s), tloc, 0.0) for s in range(MS)]
    c48 = jnp.concatenate(cols, axis=1)
    bd = (lax.broadcasted_iota(i32, (MS * BS, MS), 0) // BS
          == lax.broadcasted_iota(i32, (MS * BS, MS), 1)).astype(f32)
    s6 = jnp.dot(c48, bd, preferred_element_type=f32)
    sel_local = jnp.dot(R4T[...], s6,
                        preferred_element_type=f32).astype(i32)
    rows = (pl.program_id(0) * T
            + lax.broadcasted_iota(i32, (T, 1), 0))
    sel_ref[...] = rows * SEQ + sel_local


def _expansion_onehots():
    p = jnp.arange(TILE, dtype=jnp.int32)
    r = jnp.arange(TILE * NB, dtype=jnp.int32)
    r4 = (r[:, None] // NB == p[None, :]).astype(jnp.float32)
    return r4, r4.T


def _run_select(hid2d, Wp, bp, Wa1, ba1, Wa2, ba2, Ws1, bs1, Ws2, bs2, Bsz):
    grid = Bsz // TILE
    full = lambda shape: pl.BlockSpec(shape, lambda i: (0, 0))
    r4, r4t = _expansion_onehots()
    return pl.pallas_call(
        _select_body,
        grid=(grid,),
        in_specs=[
            pl.BlockSpec((TILE * SEQ, H), lambda i: (i, 0)),
            full((H, H)), full((1, H)),
            full((NB * H, H)), full((1, H)),
            full((H, NB)), full((1, NB)),
            full((H, H // 2)), full((1, H // 2)),
            full((H // 2, 1)), full((1, 1)),
            full((TILE * NB, TILE)), full((TILE, TILE * NB)),
        ],
        out_specs=[
            pl.BlockSpec((TILE, MS), lambda i: (i, 0)),
            pl.BlockSpec((TILE, NB), lambda i: (i, 0)),
        ],
        out_shape=[
            jax.ShapeDtypeStruct((Bsz, MS), jnp.int32),
            jax.ShapeDtypeStruct((Bsz, NB), jnp.float32),
        ],
    )(hid2d, Wp, bp, Wa1, ba1, Wa2, ba2, Ws1, bs1, Ws2, bs2, r4, r4t)


def _run_gather(hid2d, sel_flat):
    n_rows = sel_flat.shape[0]
    info = plsc.get_sparse_core_info()
    nw = info.num_cores * info.num_subcores
    per_w = n_rows // nw
    chunks = per_w // GCHUNK
    mesh = plsc.VectorSubcoreMesh(core_axis_name="c", subcore_axis_name="s")

    @functools.partial(
        pl.kernel,
        out_type=jax.ShapeDtypeStruct((n_rows, H), jnp.float32),
        mesh=mesh,
        scratch_types=[
            pltpu.VMEM((GCHUNK,), jnp.int32),
            pltpu.VMEM((GCHUNK, H), jnp.float32),
            pltpu.SemaphoreType.DMA,
        ],
        compiler_params=pltpu.CompilerParams(use_tc_tiling_on_sc=False),
    )
    def gk(hid_hbm, sel_hbm, out_hbm, idx_v, rows_v, sem):
        wid = lax.axis_index("s") * info.num_cores + lax.axis_index("c")
        base0 = wid * per_w
        for c in range(chunks):
            base = base0 + c * GCHUNK
            pltpu.sync_copy(sel_hbm.at[pl.ds(base, GCHUNK)], idx_v)
            pltpu.async_copy(hid_hbm.at[idx_v], rows_v, sem).wait()
            pltpu.sync_copy(rows_v, out_hbm.at[pl.ds(base, GCHUNK)])

    return gk(hid2d, sel_flat)


def kernel(hidden, W_pool, b_pool, W_a1, b_a1, W_a2, b_a2, W_s1, b_s1, W_s2, b_s2):
    Bsz = hidden.shape[0]
    hid2d = hidden.reshape(Bsz * SEQ, H)
    sel, ia_f = _run_select(
        hid2d, W_pool, b_pool.reshape(1, H), W_a1, b_a1.reshape(1, H),
        W_a2, b_a2.reshape(1, NB), W_s1, b_s1.reshape(1, H // 2),
        W_s2, b_s2.reshape(1, 1), Bsz)
    mem_flat = _run_gather(hid2d, sel.reshape(Bsz * MS))
    memory = mem_flat.reshape(Bsz, MS, H)
    mask = jnp.ones((Bsz, MS), hidden.dtype)
    return memory, mask, ia_f
